# baseline (speedup 1.0000x reference)
import functools

import jax
import jax.numpy as jnp
from jax import lax
from jax.experimental import pallas as pl
from jax.experimental.pallas import tpu as pltpu
from jax.experimental.pallas import tpu_sc as plsc

B = 16384
D = 16
NROWS = 1000000
LANE = 128
NC = 2
NS = 16
NW = NC * NS
ROWS_PER_W = B // NW
GRP = 16
NG = ROWS_PER_W // GRP
NSLOT = 3


def _make_sc_gather_body(rows_w):
    ng = rows_w // GRP
    nk = 2 * ng
    UNROLL = 6
    nbody = (nk - 4) // UNROLL
    tail0 = UNROLL * nbody

    def body(uid, iid, uemb_t, iemb_t, gu_t, gi_t,
             idx_u, idx_i, slab, out_u, out_i, *sems):
        wid = lax.axis_index("s") * NC + lax.axis_index("c")
        base = wid * rows_w
        iota = lax.iota(jnp.int32, D)

        pltpu.sync_copy(uid.at[pl.ds(base, rows_w)], idx_u)
        pltpu.sync_copy(iid.at[pl.ds(base, rows_w)], idx_i)

        idxs = (idx_u, idx_i)
        tabs = (uemb_t, iemb_t)
        outs = (out_u, out_i)

        def fire(k_par, slot, g):
            idx, tab = idxs[k_par], tabs[k_par]
            iv = idx[pl.ds(g * GRP, GRP)]
            cv = lax.shift_right_logical(iv, 7)
            for b in range(GRP):
                pltpu.async_copy(tab.at[:, pl.ds(cv[b] * LANE, LANE)],
                                 slab.at[slot, b], sems[slot])

        def drain(slot):
            for b in range(GRP):
                pltpu.make_async_copy(tabs[0].at[:, pl.ds(0, LANE)],
                                      slab.at[slot, b], sems[slot]).wait()

        def extract(k_par, slot, g):
            idx, out_v = idxs[k_par], outs[k_par]
            iv = idx[pl.ds(g * GRP, GRP)]
            lv = lax.bitwise_and(iv, LANE - 1)
            for b in range(GRP):
                vec = plsc.load_gather(
                    slab.at[slot, b], [iota, jnp.full((D,), lv[b], jnp.int32)])
                plsc.store_scatter(
                    out_v, [iota, jnp.full((D,), g * GRP + b, jnp.int32)], vec)

        fire(0, 0, 0)
        fire(1, 1, 0)

        def grp_body(j, carry):
            for m in range(UNROLL):
                kf = UNROLL * j + m + 2
                fire(m % 2, (m + 2) % NSLOT, kf // 2)
                drain(m % NSLOT)
                extract(m % 2, m % NSLOT, 3 * j + m // 2)
            return carry

        lax.fori_loop(0, nbody, grp_body, 0)
        for k in range(tail0, nk):
            if k + 2 < nk:
                fire((k + 2) % 2, (k + 2) % NSLOT, (k + 2) // 2)
            drain(k % NSLOT)
            extract(k % 2, k % NSLOT, k // 2)
        pltpu.sync_copy(out_u, gu_t.at[:, pl.ds(base, rows_w)])
        pltpu.sync_copy(out_i, gi_t.at[:, pl.ds(base, rows_w)])

    return body


@functools.partial(jax.jit, static_argnums=(4,))
def _sc_gather(uid, iid, uemb_t, iemb_t, nb):
    rows_w = nb // NW
    mesh = plsc.VectorSubcoreMesh(core_axis_name="c", subcore_axis_name="s")
    return pl.kernel(
        _make_sc_gather_body(rows_w),
        out_type=(
            jax.ShapeDtypeStruct((D, nb), jnp.float32),
            jax.ShapeDtypeStruct((D, nb), jnp.float32),
        ),
        mesh=mesh,
        scratch_types=[
            pltpu.VMEM((rows_w,), jnp.int32),
            pltpu.VMEM((rows_w,), jnp.int32),
            pltpu.VMEM((NSLOT, GRP, D, LANE), jnp.float32),
            pltpu.VMEM((D, rows_w), jnp.float32),
            pltpu.VMEM((D, rows_w), jnp.float32),
        ] + [pltpu.SemaphoreType.DMA] * NSLOT,
        compiler_params=pltpu.CompilerParams(needs_layout_passes=False),
    )(uid, iid, uemb_t, iemb_t)


BLK = 4096


def _mlp_body(gu_t, gi_t, w1u, w1i, b1, w2, b2, w3, b3, out):
    dn = (((0,), (0,)), ((), ()))
    h = lax.dot_general(w1u[...], gu_t[...], dn,
                        preferred_element_type=jnp.float32)
    h = h + lax.dot_general(w1i[...], gi_t[...], dn,
                            preferred_element_type=jnp.float32)
    h = jnp.maximum(h + b1[...], 0.0)
    h = jnp.maximum(
        lax.dot_general(w2[...], h, dn, preferred_element_type=jnp.float32)
        + b2[...], 0.0)
    o = lax.dot_general(w3[...], h, dn,
                        preferred_element_type=jnp.float32) + b3[...]
    out[...] = jnp.reshape(o, (BLK,))


def _mlp(gu_t, gi_t, W1, b1, W2, b2, W3, b3):
    nb = gu_t.shape[1]
    w1u = W1[:D, :]
    w1i = W1[D:, :]
    b1r = jnp.reshape(b1, (-1, 1))
    b2r = jnp.reshape(b2, (-1, 1))
    b3r = jnp.reshape(b3, (1, 1))
    grid = (nb // BLK,)
    return pl.pallas_call(
        _mlp_body,
        grid=grid,
        in_specs=[
            pl.BlockSpec((D, BLK), lambda i: (0, i)),
            pl.BlockSpec((D, BLK), lambda i: (0, i)),
            pl.BlockSpec(w1u.shape, lambda i: (0, 0)),
            pl.BlockSpec(w1i.shape, lambda i: (0, 0)),
            pl.BlockSpec(b1r.shape, lambda i: (0, 0)),
            pl.BlockSpec(W2.shape, lambda i: (0, 0)),
            pl.BlockSpec(b2r.shape, lambda i: (0, 0)),
            pl.BlockSpec(W3.shape, lambda i: (0, 0)),
            pl.BlockSpec(b3r.shape, lambda i: (0, 0)),
        ],
        out_specs=pl.BlockSpec((BLK,), lambda i: (i,)),
        out_shape=jax.ShapeDtypeStruct((nb,), jnp.float32),
    )(gu_t, gi_t, w1u, w1i, b1r, W2, b2r, W3, b3r)


NPHASE = 2


def kernel(user_id, item_id, user_emb, item_emb, W1, b1, W2, b2, W3, b3):
    uid = user_id.astype(jnp.int32)
    iid = item_id.astype(jnp.int32)
    ut, it = user_emb.T, item_emb.T
    hb = B // NPHASE
    flats = []
    for p in range(NPHASE):
        gu_t, gi_t = _sc_gather(uid[p * hb:(p + 1) * hb],
                                iid[p * hb:(p + 1) * hb], ut, it, hb)
        flats.append(_mlp(gu_t, gi_t, W1, b1, W2, b2, W3, b3))
    return jnp.reshape(jnp.concatenate(flats), (B, 1))


# speedup vs baseline: 1.0954x; 1.0954x over previous
import functools

import jax
import jax.numpy as jnp
from jax import lax
from jax.experimental import pallas as pl
from jax.experimental.pallas import tpu as pltpu
from jax.experimental.pallas import tpu_sc as plsc

B = 16384
D = 16
NROWS = 1000000
LANE = 128
NC = 2
NS = 16
NW = NC * NS
ROWS_PER_W = B // NW
GRP = 16
NG = ROWS_PER_W // GRP
NSLOT = 3


def _make_sc_gather_body(rows_w):
    ng = rows_w // GRP
    nk = 2 * ng
    UNROLL = 6
    nbody = (nk - 4) // UNROLL
    tail0 = UNROLL * nbody

    def body(uid, iid, uemb_t, iemb_t, gu_t, gi_t,
             idx_u, idx_i, slab, out_u, out_i, *sems):
        wid = lax.axis_index("s") * NC + lax.axis_index("c")
        base = wid * rows_w
        iota = lax.iota(jnp.int32, D)

        pltpu.sync_copy(uid.at[pl.ds(base, rows_w)], idx_u)
        pltpu.sync_copy(iid.at[pl.ds(base, rows_w)], idx_i)

        idxs = (idx_u, idx_i)
        tabs = (uemb_t, iemb_t)
        outs = (out_u, out_i)

        def fire(k_par, slot, g):
            idx, tab = idxs[k_par], tabs[k_par]
            iv = idx[pl.ds(g * GRP, GRP)]
            cv = lax.shift_right_logical(iv, 7)
            for b in range(GRP):
                pltpu.async_copy(tab.at[:, pl.ds(cv[b] * LANE, LANE)],
                                 slab.at[slot, b], sems[slot])

        def drain(slot):
            for b in range(GRP):
                pltpu.make_async_copy(tabs[0].at[:, pl.ds(0, LANE)],
                                      slab.at[slot, b], sems[slot]).wait()

        def extract(k_par, slot, g):
            idx, out_v = idxs[k_par], outs[k_par]
            iv = idx[pl.ds(g * GRP, GRP)]
            lv = lax.bitwise_and(iv, LANE - 1)
            for b in range(GRP):
                vec = plsc.load_gather(
                    slab.at[slot, b], [iota, jnp.full((D,), lv[b], jnp.int32)])
                plsc.store_scatter(
                    out_v, [iota, jnp.full((D,), g * GRP + b, jnp.int32)], vec)

        fire(0, 0, 0)
        fire(1, 1, 0)

        def grp_body(j, carry):
            for m in range(UNROLL):
                kf = UNROLL * j + m + 2
                fire(m % 2, (m + 2) % NSLOT, kf // 2)
                drain(m % NSLOT)
                extract(m % 2, m % NSLOT, 3 * j + m // 2)
            return carry

        lax.fori_loop(0, nbody, grp_body, 0)
        for k in range(tail0, nk):
            if k + 2 < nk:
                fire((k + 2) % 2, (k + 2) % NSLOT, (k + 2) // 2)
            drain(k % NSLOT)
            extract(k % 2, k % NSLOT, k // 2)
        pltpu.sync_copy(out_u, gu_t.at[:, pl.ds(base, rows_w)])
        pltpu.sync_copy(out_i, gi_t.at[:, pl.ds(base, rows_w)])

    return body


@functools.partial(jax.jit, static_argnums=(4,))
def _sc_gather(uid, iid, uemb_t, iemb_t, nb):
    rows_w = nb // NW
    mesh = plsc.VectorSubcoreMesh(core_axis_name="c", subcore_axis_name="s")
    return pl.kernel(
        _make_sc_gather_body(rows_w),
        out_type=(
            jax.ShapeDtypeStruct((D, nb), jnp.float32),
            jax.ShapeDtypeStruct((D, nb), jnp.float32),
        ),
        mesh=mesh,
        scratch_types=[
            pltpu.VMEM((rows_w,), jnp.int32),
            pltpu.VMEM((rows_w,), jnp.int32),
            pltpu.VMEM((NSLOT, GRP, D, LANE), jnp.float32),
            pltpu.VMEM((D, rows_w), jnp.float32),
            pltpu.VMEM((D, rows_w), jnp.float32),
        ] + [pltpu.SemaphoreType.DMA] * NSLOT,
        compiler_params=pltpu.CompilerParams(needs_layout_passes=False),
    )(uid, iid, uemb_t, iemb_t)


BLK = 4096


def _mlp_body(gu_t, gi_t, w1u, w1i, b1, w2, b2, w3, b3, out):
    dn = (((0,), (0,)), ((), ()))
    h = lax.dot_general(w1u[...], gu_t[...], dn,
                        preferred_element_type=jnp.float32)
    h = h + lax.dot_general(w1i[...], gi_t[...], dn,
                            preferred_element_type=jnp.float32)
    h = jnp.maximum(h + b1[...], 0.0)
    h = jnp.maximum(
        lax.dot_general(w2[...], h, dn, preferred_element_type=jnp.float32)
        + b2[...], 0.0)
    o = lax.dot_general(w3[...], h, dn,
                        preferred_element_type=jnp.float32) + b3[...]
    out[...] = jnp.reshape(o, (BLK,))


def _mlp(gu_t, gi_t, W1, b1, W2, b2, W3, b3):
    nb = gu_t.shape[1]
    w1u = W1[:D, :]
    w1i = W1[D:, :]
    b1r = jnp.reshape(b1, (-1, 1))
    b2r = jnp.reshape(b2, (-1, 1))
    b3r = jnp.reshape(b3, (1, 1))
    grid = (nb // BLK,)
    return pl.pallas_call(
        _mlp_body,
        grid=grid,
        in_specs=[
            pl.BlockSpec((D, BLK), lambda i: (0, i)),
            pl.BlockSpec((D, BLK), lambda i: (0, i)),
            pl.BlockSpec(w1u.shape, lambda i: (0, 0)),
            pl.BlockSpec(w1i.shape, lambda i: (0, 0)),
            pl.BlockSpec(b1r.shape, lambda i: (0, 0)),
            pl.BlockSpec(W2.shape, lambda i: (0, 0)),
            pl.BlockSpec(b2r.shape, lambda i: (0, 0)),
            pl.BlockSpec(W3.shape, lambda i: (0, 0)),
            pl.BlockSpec(b3r.shape, lambda i: (0, 0)),
        ],
        out_specs=pl.BlockSpec((BLK,), lambda i: (i,)),
        out_shape=jax.ShapeDtypeStruct((nb,), jnp.float32),
    )(gu_t, gi_t, w1u, w1i, b1r, W2, b2r, W3, b3r)


NPHASE = 1


def kernel(user_id, item_id, user_emb, item_emb, W1, b1, W2, b2, W3, b3):
    uid = user_id.astype(jnp.int32)
    iid = item_id.astype(jnp.int32)
    ut, it = user_emb.T, item_emb.T
    hb = B // NPHASE
    flats = []
    for p in range(NPHASE):
        gu_t, gi_t = _sc_gather(uid[p * hb:(p + 1) * hb],
                                iid[p * hb:(p + 1) * hb], ut, it, hb)
        flats.append(_mlp(gu_t, gi_t, W1, b1, W2, b2, W3, b3))
    return jnp.reshape(jnp.concatenate(flats), (B, 1))


# speedup vs baseline: 1.0982x; 1.0026x over previous
import functools

import jax
import jax.numpy as jnp
from jax import lax
from jax.experimental import pallas as pl
from jax.experimental.pallas import tpu as pltpu
from jax.experimental.pallas import tpu_sc as plsc

B = 16384
D = 16
LANE = 128
NC = 2
NS = 16
NW = NC * NS
GRP = 16
NSLOT = 3


def _make_sc_gather_body(rows_w):
    ng = rows_w // GRP
    nk = 2 * ng
    UNROLL = 6
    nbody = (nk - 4) // UNROLL
    tail0 = UNROLL * nbody

    def body(uid, iid, uemb_t, iemb_t, gu_t, gi_t,
             idx_u, idx_i, slab, out_u, out_i, *sems):
        wid = lax.axis_index("s") * NC + lax.axis_index("c")
        base = wid * rows_w
        iota = lax.iota(jnp.int32, D)

        pltpu.sync_copy(uid.at[pl.ds(base, rows_w)], idx_u)
        pltpu.sync_copy(iid.at[pl.ds(base, rows_w)], idx_i)

        idxs = (idx_u, idx_i)
        tabs = (uemb_t, iemb_t)
        outs = (out_u, out_i)

        def fire(k_par, slot, g):
            idx, tab = idxs[k_par], tabs[k_par]
            iv = idx[pl.ds(g * GRP, GRP)]
            cv = lax.shift_right_logical(iv, 7)
            for b in range(GRP):
                pltpu.async_copy(tab.at[:, pl.ds(cv[b] * LANE, LANE)],
                                 slab.at[slot, b], sems[slot])

        def drain(slot):
            for b in range(GRP):
                pltpu.make_async_copy(tabs[0].at[:, pl.ds(0, LANE)],
                                      slab.at[slot, b], sems[slot]).wait()

        def extract(k_par, slot, g):
            idx, out_v = idxs[k_par], outs[k_par]
            iv = idx[pl.ds(g * GRP, GRP)]
            lv = lax.bitwise_and(iv, LANE - 1)
            for b in range(GRP):
                vec = plsc.load_gather(
                    slab.at[slot, b], [iota, jnp.full((D,), lv[b], jnp.int32)])
                plsc.store_scatter(
                    out_v, [iota, jnp.full((D,), g * GRP + b, jnp.int32)], vec)

        fire(0, 0, 0)
        fire(1, 1, 0)

        def grp_body(j, carry):
            for m in range(UNROLL):
                kf = UNROLL * j + m + 2
                fire(m % 2, (m + 2) % NSLOT, kf // 2)
                drain(m % NSLOT)
                extract(m % 2, m % NSLOT, 3 * j + m // 2)
            return carry

        lax.fori_loop(0, nbody, grp_body, 0)
        for k in range(tail0, nk):
            if k + 2 < nk:
                fire((k + 2) % 2, (k + 2) % NSLOT, (k + 2) // 2)
            drain(k % NSLOT)
            extract(k % 2, k % NSLOT, k // 2)
        pltpu.sync_copy(out_u, gu_t.at[:, pl.ds(base, rows_w)])
        pltpu.sync_copy(out_i, gi_t.at[:, pl.ds(base, rows_w)])

    return body


@functools.partial(jax.jit, static_argnums=(4,))
def _sc_gather(uid, iid, uemb_t, iemb_t, nb):
    rows_w = nb // NW
    mesh = plsc.VectorSubcoreMesh(core_axis_name="c", subcore_axis_name="s")
    return pl.kernel(
        _make_sc_gather_body(rows_w),
        out_type=(
            jax.ShapeDtypeStruct((D, nb), jnp.float32),
            jax.ShapeDtypeStruct((D, nb), jnp.float32),
        ),
        mesh=mesh,
        scratch_types=[
            pltpu.VMEM((rows_w,), jnp.int32),
            pltpu.VMEM((rows_w,), jnp.int32),
            pltpu.VMEM((NSLOT, GRP, D, LANE), jnp.float32),
            pltpu.VMEM((D, rows_w), jnp.float32),
            pltpu.VMEM((D, rows_w), jnp.float32),
        ] + [pltpu.SemaphoreType.DMA] * NSLOT,
        compiler_params=pltpu.CompilerParams(needs_layout_passes=False),
    )(uid, iid, uemb_t, iemb_t)


BLK = 4096


def _mlp_body(gu_t, gi_t, w1u, w1i, b1, w2, b2, w3, b3, out):
    dn = (((0,), (0,)), ((), ()))
    h = lax.dot_general(w1u[...], gu_t[...], dn,
                        preferred_element_type=jnp.float32)
    h = h + lax.dot_general(w1i[...], gi_t[...], dn,
                            preferred_element_type=jnp.float32)
    h = jnp.maximum(h + b1[...], 0.0)
    h = jnp.maximum(
        lax.dot_general(w2[...], h, dn, preferred_element_type=jnp.float32)
        + b2[...], 0.0)
    o = lax.dot_general(w3[...], h, dn,
                        preferred_element_type=jnp.float32) + b3[...]
    out[...] = jnp.reshape(o, (BLK,))


def _mlp(gu_t, gi_t, W1, b1, W2, b2, W3, b3):
    nb = gu_t.shape[1]
    w1u = W1[:D, :]
    w1i = W1[D:, :]
    b1r = jnp.reshape(b1, (-1, 1))
    b2r = jnp.reshape(b2, (-1, 1))
    b3r = jnp.reshape(b3, (1, 1))
    grid = (nb // BLK,)
    return pl.pallas_call(
        _mlp_body,
        grid=grid,
        in_specs=[
            pl.BlockSpec((D, BLK), lambda i: (0, i)),
            pl.BlockSpec((D, BLK), lambda i: (0, i)),
            pl.BlockSpec(w1u.shape, lambda i: (0, 0)),
            pl.BlockSpec(w1i.shape, lambda i: (0, 0)),
            pl.BlockSpec(b1r.shape, lambda i: (0, 0)),
            pl.BlockSpec(W2.shape, lambda i: (0, 0)),
            pl.BlockSpec(b2r.shape, lambda i: (0, 0)),
            pl.BlockSpec(W3.shape, lambda i: (0, 0)),
            pl.BlockSpec(b3r.shape, lambda i: (0, 0)),
        ],
        out_specs=pl.BlockSpec((BLK,), lambda i: (i,)),
        out_shape=jax.ShapeDtypeStruct((nb,), jnp.float32),
    )(gu_t, gi_t, w1u, w1i, b1r, W2, b2r, W3, b3r)


NPHASE = 1


def kernel(user_id, item_id, user_emb, item_emb, W1, b1, W2, b2, W3, b3):
    uid = user_id.astype(jnp.int32)
    iid = item_id.astype(jnp.int32)
    ut, it = user_emb.T, item_emb.T
    hb = B // NPHASE
    flats = []
    for p in range(NPHASE):
        gu_t, gi_t = _sc_gather(uid[p * hb:(p + 1) * hb],
                                iid[p * hb:(p + 1) * hb], ut, it, hb)
        flats.append(_mlp(gu_t, gi_t, W1, b1, W2, b2, W3, b3))
    return jnp.reshape(jnp.concatenate(flats), (B, 1))
